# Initial kernel scaffold; baseline (speedup 1.0000x reference)
import functools

import jax
import jax.numpy as jnp
from jax import lax
from jax.experimental import pallas as pl
from jax.experimental.pallas import tpu as pltpu
from jax.experimental.pallas import tpu_sc as plsc

NN = 10000
ET = 160000 + NN
NC, NS, LN = 2, 16, 16
NWORK = NC * NS
BB = 128
NBATCH = 42
CHUNK = BB * NBATCH
EP = CHUNK * NWORK
C1W = 80
C2W = 144
NROW = NN // NS
NRC = 125

_i32 = jnp.int32
_f32 = jnp.float32


def _tc_a_body(x_ref, w1_ref, ms_ref, md_ref, t1_ref, ad_ref):
    h1 = jnp.dot(x_ref[...], w1_ref[...], preferred_element_type=_f32)
    as1 = jnp.dot(h1, ms_ref[...], preferred_element_type=_f32)
    ad1 = jnp.dot(h1, md_ref[...], preferred_element_type=_f32)
    t1_ref[:, 0:64] = h1
    t1_ref[:, 64:72] = as1
    t1_ref[:, 72:80] = as1
    ad_ref[...] = ad1


def _tc_a(x, W1, ms, md):
    blk = 1000
    return pl.pallas_call(
        _tc_a_body,
        grid=(NN // blk,),
        in_specs=[
            pl.BlockSpec((blk, 256), lambda i: (i, 0)),
            pl.BlockSpec((256, 64), lambda i: (0, 0)),
            pl.BlockSpec((64, 8), lambda i: (0, 0)),
            pl.BlockSpec((64, 8), lambda i: (0, 0)),
        ],
        out_specs=[
            pl.BlockSpec((blk, C1W), lambda i: (i, 0)),
            pl.BlockSpec((blk, 8), lambda i: (i, 0)),
        ],
        out_shape=[
            jax.ShapeDtypeStruct((NN, C1W), _f32),
            jax.ShapeDtypeStruct((NN, 8), _f32),
        ],
    )(x, W1, ms, md)


def _tc_b_body(a_ref, b_ref, b1_ref, w2_ref, m2_ref, r_ref, h2_ref, aa_ref):
    msg = a_ref[:, 0:64] + b_ref[:, 0:64]
    den = a_ref[:, 64:72] + b_ref[:, 64:72]
    denx = jnp.dot(den, r_ref[...], preferred_element_type=_f32)
    h = msg / denx + b1_ref[...]
    h = jnp.where(h > 0.0, h, jnp.exp(h) - 1.0)
    h2 = jnp.dot(h, w2_ref[...], preferred_element_type=_f32)
    aa = jnp.dot(h2, m2_ref[...], preferred_element_type=_f32)
    h2_ref[...] = h2
    aa_ref[...] = aa


def _tc_b(acc_a, acc_b, b1, W2, m2, rexp):
    blk = 1000
    return pl.pallas_call(
        _tc_b_body,
        grid=(NN // blk,),
        in_specs=[
            pl.BlockSpec((blk, C1W), lambda i: (i, 0)),
            pl.BlockSpec((blk, C1W), lambda i: (i, 0)),
            pl.BlockSpec((1, 64), lambda i: (0, 0)),
            pl.BlockSpec((64, 256), lambda i: (0, 0)),
            pl.BlockSpec((256, 2), lambda i: (0, 0)),
            pl.BlockSpec((8, 64), lambda i: (0, 0)),
        ],
        out_specs=[
            pl.BlockSpec((blk, 256), lambda i: (i, 0)),
            pl.BlockSpec((blk, 2), lambda i: (i, 0)),
        ],
        out_shape=[
            jax.ShapeDtypeStruct((NN, 256), _f32),
            jax.ShapeDtypeStruct((NN, 2), _f32),
        ],
    )(acc_a, acc_b, b1, W2, m2, rexp)


def _tc_c_body(a_ref, b_ref, b2_ref, o_ref):
    oa = a_ref[:, 0:128] / a_ref[:, 128:129]
    ob = b_ref[:, 0:128] / b_ref[:, 128:129]
    o = jnp.concatenate([oa, ob], axis=1) + b2_ref[...]
    m = jnp.max(o, axis=1, keepdims=True)
    lse = m + jnp.log(jnp.sum(jnp.exp(o - m), axis=1, keepdims=True))
    o_ref[...] = o - lse


def _tc_c(acc_a, acc_b, b2):
    blk = 1000
    return pl.pallas_call(
        _tc_c_body,
        grid=(NN // blk,),
        in_specs=[
            pl.BlockSpec((blk, C2W), lambda i: (i, 0)),
            pl.BlockSpec((blk, C2W), lambda i: (i, 0)),
            pl.BlockSpec((1, 256), lambda i: (0, 0)),
        ],
        out_specs=pl.BlockSpec((blk, 256), lambda i: (i, 0)),
        out_shape=jax.ShapeDtypeStruct((NN, 256), _f32),
    )(acc_a, acc_b, b2)


def _zero_acc(msgbuf, acc_s, width, sub):
    nst = width // LN

    def zrow(i, _):
        r = i // nst
        j = i % nst
        msgbuf[r, pl.ds(j * LN, LN)] = jnp.zeros((LN,), _f32)
        return 0

    lax.fori_loop(0, NRC * nst, zrow, 0)
    for k in range(NROW // NRC):
        pltpu.sync_copy(msgbuf.at[pl.ds(0, NRC)],
                        acc_s.at[pl.ds(sub * NROW + k * NRC, NRC)])


def _copy_out(msgbuf, acc_s, out_hbm, core, sub):
    for k in range(NROW // NRC):
        rows = pl.ds(sub * NROW + k * NRC, NRC)
        pltpu.sync_copy(acc_s.at[rows], msgbuf.at[pl.ds(0, NRC)])
        pltpu.sync_copy(msgbuf.at[pl.ds(0, NRC)], out_hbm.at[core, rows])


def _sc1_body(t1_hbm, ad_hbm, src_hbm, dst_hbm, out_hbm,
              adt_l, src_l, dst_l, rowbuf, msgbuf, pbuf, acc_s, sem):
    core = lax.axis_index("c")
    sub = lax.axis_index("s")
    g = core * NS + sub
    iota = lax.iota(_i32, LN)
    _zero_acc(msgbuf, acc_s, C1W, sub)
    plsc.subcore_barrier()

    pltpu.sync_copy(ad_hbm, adt_l)
    pltpu.sync_copy(src_hbm.at[g], src_l)
    pltpu.sync_copy(dst_hbm.at[g], dst_l)
    base = g * CHUNK

    ovecs = [j * 2 + jnp.where(iota >= 8, 1, 0) for j in range(4)]
    omod = iota % 8

    def batch(b, _):
        pltpu.async_copy(t1_hbm.at[src_l.at[b]], rowbuf, sem).wait()

        def group(g2, _):
            dstv = dst_l[b, 0, pl.ds(g2 * LN, LN)]
            rowi = g2 * LN + iota
            valid = (base + b * BB + rowi) < ET
            for h in range(8):
                hf = jnp.full((LN,), h, _i32)
                a_s = plsc.load_gather(rowbuf, [rowi, hf + 64])
                a_d = plsc.load_gather(adt_l, [dstv, hf])
                al = a_s + a_d
                al = jnp.where(al > 0.0, al, 0.2 * al)
                p = jnp.where(valid, jnp.exp(al), 0.0)
                plsc.store_scatter(pbuf, [iota, hf], p)
            for e in range(16):
                er = g2 * LN + e
                ef = jnp.full((LN,), e, _i32)
                for j in range(4):
                    pe = plsc.load_gather(pbuf, [ef, ovecs[j]])
                    msgbuf[er, pl.ds(j * LN, LN)] = (
                        rowbuf[er, pl.ds(j * LN, LN)] * pe)
                pt = plsc.load_gather(pbuf, [ef, omod])
                msgbuf[er, pl.ds(64, LN)] = pt
            return 0

        lax.fori_loop(0, BB // LN, group, 0)
        pltpu.sync_copy(msgbuf, acc_s.at[dst_l.at[b]], add=True)
        return 0

    lax.fori_loop(0, NBATCH, batch, 0)
    plsc.subcore_barrier()
    _copy_out(msgbuf, acc_s, out_hbm, core, sub)


def _sc_phase1(t1, adt, srcg, dstg):
    mesh = plsc.VectorSubcoreMesh(
        core_axis_name="c", subcore_axis_name="s",
        num_cores=NC, num_subcores=NS)
    f = functools.partial(
        pl.kernel,
        out_type=jax.ShapeDtypeStruct((NC, NN, C1W), _f32),
        mesh=mesh,
        scratch_types=[
            pltpu.VMEM((NN, 8), _f32),
            pltpu.VMEM((NBATCH, BB), _i32),
            pltpu.VMEM((NBATCH, 1, BB), _i32),
            pltpu.VMEM((BB, C1W), _f32),
            pltpu.VMEM((BB, C1W), _f32),
            pltpu.VMEM((LN, 8), _f32),
            pltpu.VMEM_SHARED((NN, C1W), _f32),
            pltpu.SemaphoreType.DMA,
        ],
    )(_sc1_body)
    return f(t1, adt, srcg, dstg)


def _sc2_body(h2_hbm, aa_hbm, src_hbm, dst_hbm, out_hbm,
              aa_l, src_l, idx2_l, dst_l, rowbuf, msgbuf, qbuf, acc_s, sem):
    core = lax.axis_index("c")
    sub = lax.axis_index("s")
    iota = lax.iota(_i32, LN)
    _zero_acc(msgbuf, acc_s, C2W, sub)
    plsc.subcore_barrier()

    pltpu.sync_copy(aa_hbm, aa_l)
    zf = jnp.zeros((LN,), _i32)
    of = jnp.full((LN,), 1, _i32)

    def chunk(ci, _):
        g = sub * 2 + ci
        pltpu.sync_copy(src_hbm.at[g], src_l)
        pltpu.sync_copy(dst_hbm.at[g], dst_l)
        base = g * CHUNK

        def mkidx(i, _):
            b = i // (BB // LN)
            k = i % (BB // LN)
            v = src_l[b, pl.ds(k * LN, LN)]
            idx2_l[b, pl.ds(k * LN, LN)] = v * 2 + core
            return 0

        lax.fori_loop(0, NBATCH * (BB // LN), mkidx, 0)

        def batch(b, _):
            pltpu.async_copy(h2_hbm.at[idx2_l.at[b]], rowbuf, sem).wait()

            def group(g2, _):
                srcv = src_l[b, pl.ds(g2 * LN, LN)]
                dstv = dst_l[b, 0, pl.ds(g2 * LN, LN)]
                rowi = g2 * LN + iota
                valid = (base + b * BB + rowi) < ET
                a_s = plsc.load_gather(aa_l, [srcv, zf])
                a_d = plsc.load_gather(aa_l, [dstv, of])
                al = a_s + a_d
                al = jnp.where(al > 0.0, al, 0.2 * al)
                q = jnp.where(valid, jnp.exp(al), 0.0)
                qbuf[...] = q
                for e in range(16):
                    er = g2 * LN + e
                    qe = plsc.load_gather(qbuf, [jnp.full((LN,), e, _i32)])
                    for j in range(8):
                        msgbuf[er, pl.ds(j * LN, LN)] = (
                            rowbuf[er, pl.ds(j * LN, LN)] * qe)
                    msgbuf[er, pl.ds(128, LN)] = qe
                return 0

            lax.fori_loop(0, BB // LN, group, 0)
            pltpu.sync_copy(msgbuf, acc_s.at[dst_l.at[b]], add=True)
            return 0

        lax.fori_loop(0, NBATCH, batch, 0)
        return 0

    lax.fori_loop(0, 2, chunk, 0)
    plsc.subcore_barrier()
    _copy_out(msgbuf, acc_s, out_hbm, core, sub)


def _sc_phase2(h2pk, aa, srcg, dstg):
    mesh = plsc.VectorSubcoreMesh(
        core_axis_name="c", subcore_axis_name="s",
        num_cores=NC, num_subcores=NS)
    f = functools.partial(
        pl.kernel,
        out_type=jax.ShapeDtypeStruct((NC, NN, C2W), _f32),
        mesh=mesh,
        scratch_types=[
            pltpu.VMEM((NN, 2), _f32),
            pltpu.VMEM((NBATCH, BB), _i32),
            pltpu.VMEM((NBATCH, BB), _i32),
            pltpu.VMEM((NBATCH, 1, BB), _i32),
            pltpu.VMEM((BB, 128), _f32),
            pltpu.VMEM((BB, C2W), _f32),
            pltpu.VMEM((LN,), _f32),
            pltpu.VMEM_SHARED((NN, C2W), _f32),
            pltpu.SemaphoreType.DMA,
        ],
    )(_sc2_body)
    return f(h2pk, aa, srcg, dstg)


@jax.jit
def kernel(x, edge_index, W1, a_src1, a_dst1, b1, W2, a_src2, a_dst2, b2):
    loop = jnp.arange(NN, dtype=_i32)
    src = jnp.concatenate([edge_index[0].astype(_i32), loop])
    dst = jnp.concatenate([edge_index[1].astype(_i32), loop])
    pad = jnp.zeros((EP - ET,), _i32)
    srcg = jnp.concatenate([src, pad]).reshape(NWORK, NBATCH, BB)
    dstg = jnp.concatenate([dst, pad]).reshape(NWORK, NBATCH, 1, BB)

    diag = jnp.repeat(jnp.eye(8, dtype=_f32), 8, axis=0)
    ms = diag * a_src1.reshape(64)[:, None]
    md = diag * a_dst1.reshape(64)[:, None]
    rexp = jnp.repeat(jnp.eye(8, dtype=_f32), 8, axis=1)
    m2 = jnp.stack([a_src2.reshape(256), a_dst2.reshape(256)], axis=1)

    t1, adt = _tc_a(x, W1, ms, md)
    acc1 = _sc_phase1(t1, adt, srcg, dstg)
    h2, aa = _tc_b(acc1[0], acc1[1], b1.reshape(1, 64), W2, m2, rexp)
    h2pk = h2.reshape(2 * NN, 128)
    acc2 = _sc_phase2(h2pk, aa, srcg, dstg)
    return _tc_c(acc2[0], acc2[1], b2.reshape(1, 256))


# speedup vs baseline: 16.9240x; 16.9240x over previous
import functools

import jax
import jax.numpy as jnp
from jax import lax
from jax.experimental import pallas as pl
from jax.experimental.pallas import tpu as pltpu
from jax.experimental.pallas import tpu_sc as plsc

NN = 10000
ET = 160000 + NN
NC, NS, LN = 2, 16, 16
NWORK = NC * NS
BB = 128
NBATCH = 42
CHUNK = BB * NBATCH
EP = CHUNK * NWORK
T1W = 32
C1W = 16
C2W = 32
NROW = NN // NS
NRC = 125

_i32 = jnp.int32
_f32 = jnp.float32


def _tc_a_body(x_ref, w1_ref, ms_ref, md_ref, t1_ref, ad_ref):
    h1 = jnp.dot(x_ref[...], w1_ref[...], preferred_element_type=_f32)
    as1 = jnp.dot(h1, ms_ref[...], preferred_element_type=_f32)
    ad1 = jnp.dot(h1, md_ref[...], preferred_element_type=_f32)
    blk = h1.shape[0]
    zpad = jnp.zeros((blk, T1W - 18), _f32)
    for k in range(4):
        t1_ref[:, k, :] = jnp.concatenate(
            [h1[:, k * 16:(k + 1) * 16], as1[:, k * 2:k * 2 + 2], zpad],
            axis=1)
    for c in range(NC):
        ad_ref[c, :, :] = ad1[:, c * 4:(c + 1) * 4]


def _tc_a(x, W1, ms, md):
    blk = 1000
    return pl.pallas_call(
        _tc_a_body,
        grid=(NN // blk,),
        in_specs=[
            pl.BlockSpec((blk, 256), lambda i: (i, 0)),
            pl.BlockSpec((256, 64), lambda i: (0, 0)),
            pl.BlockSpec((64, 8), lambda i: (0, 0)),
            pl.BlockSpec((64, 8), lambda i: (0, 0)),
        ],
        out_specs=[
            pl.BlockSpec((blk, 4, T1W), lambda i: (i, 0, 0)),
            pl.BlockSpec((NC, blk, 4), lambda i: (0, i, 0)),
        ],
        out_shape=[
            jax.ShapeDtypeStruct((NN, 4, T1W), _f32),
            jax.ShapeDtypeStruct((NC, NN, 4), _f32),
        ],
    )(x, W1, ms, md)


def _tc_b_body(acc_ref, dn_ref, b1_ref, w2_ref, m2_ref, r_ref,
               h2_ref, aa_ref):
    msg = jnp.concatenate([acc_ref[k] for k in range(4)], axis=1)
    den = jnp.concatenate([dn_ref[0, :, 0:4], dn_ref[1, :, 0:4]], axis=1)
    denx = jnp.dot(den, r_ref[...], preferred_element_type=_f32)
    h = msg / denx + b1_ref[...]
    h = jnp.where(h > 0.0, h, jnp.exp(h) - 1.0)
    h2 = jnp.dot(h, w2_ref[...], preferred_element_type=_f32)
    aa = jnp.dot(h2, m2_ref[...], preferred_element_type=_f32)
    h2_ref[...] = h2
    aa_ref[...] = aa


def _tc_b(acc1, dn1, b1, W2, m2, rexp):
    blk = 1000
    return pl.pallas_call(
        _tc_b_body,
        grid=(NN // blk,),
        in_specs=[
            pl.BlockSpec((4, blk, C1W), lambda i: (0, i, 0)),
            pl.BlockSpec((NC, blk, 8), lambda i: (0, i, 0)),
            pl.BlockSpec((1, 64), lambda i: (0, 0)),
            pl.BlockSpec((64, 256), lambda i: (0, 0)),
            pl.BlockSpec((256, 2), lambda i: (0, 0)),
            pl.BlockSpec((8, 64), lambda i: (0, 0)),
        ],
        out_specs=[
            pl.BlockSpec((blk, 256), lambda i: (i, 0)),
            pl.BlockSpec((blk, 2), lambda i: (i, 0)),
        ],
        out_shape=[
            jax.ShapeDtypeStruct((NN, 256), _f32),
            jax.ShapeDtypeStruct((NN, 2), _f32),
        ],
    )(acc1, dn1, b1, W2, m2, rexp)


def _tc_c_body(acc_ref, da_ref, db_ref, b2_ref, o_ref):
    oa = jnp.concatenate([acc_ref[k] for k in range(4)], axis=1) / (
        da_ref[...] * 0.25)
    ob = jnp.concatenate([acc_ref[k] for k in range(4, 8)], axis=1) / (
        db_ref[...] * 0.25)
    o = jnp.concatenate([oa, ob], axis=1) + b2_ref[...]
    m = jnp.max(o, axis=1, keepdims=True)
    lse = m + jnp.log(jnp.sum(jnp.exp(o - m), axis=1, keepdims=True))
    o_ref[...] = o - lse


def _tc_c(acc2, den_a, den_b, b2):
    blk = 1000
    return pl.pallas_call(
        _tc_c_body,
        grid=(NN // blk,),
        in_specs=[
            pl.BlockSpec((8, blk, C2W), lambda i: (0, i, 0)),
            pl.BlockSpec((blk, 1), lambda i: (i, 0)),
            pl.BlockSpec((blk, 1), lambda i: (i, 0)),
            pl.BlockSpec((1, 256), lambda i: (0, 0)),
        ],
        out_specs=pl.BlockSpec((blk, 256), lambda i: (i, 0)),
        out_shape=jax.ShapeDtypeStruct((NN, 256), _f32),
    )(acc2, den_a, den_b, b2)


def _zero_buf(msgbuf, width):
    offs = list(range(0, width - LN + 1, LN))
    if width % LN:
        offs.append(width - LN)

    def zrow(r, _):
        for o in offs:
            msgbuf[r, pl.ds(o, LN)] = jnp.zeros((LN,), _f32)
        return 0

    lax.fori_loop(0, NRC, zrow, 0)


def _zero_acc(msgbuf, acc_s, sub):
    for k in range(NROW // NRC):
        pltpu.sync_copy(msgbuf.at[pl.ds(0, NRC)],
                        acc_s.at[pl.ds(sub * NROW + k * NRC, NRC)])


def _copy_out(msgbuf, acc_s, out_hbm, core, sub):
    for k in range(NROW // NRC):
        rows = pl.ds(sub * NROW + k * NRC, NRC)
        pltpu.sync_copy(acc_s.at[rows], msgbuf.at[pl.ds(0, NRC)])
        pltpu.sync_copy(msgbuf.at[pl.ds(0, NRC)], out_hbm.at[core, rows])


def _mk_idx2(src_l, idx2_l, core):
    def mkidx(i, _):
        v = src_l[pl.ds(i * LN, LN)]
        idx2_l[pl.ds(i * LN, LN)] = v * 2 + core
        return 0

    lax.fori_loop(0, CHUNK // LN, mkidx, 0)


def _sc1_body(t1_hbm, ad_hbm, src_hbm, dst_hbm, out_hbm, dn_hbm,
              adt_l, src_l, idx2_l, dst_l, rowbuf, msgbuf, pbuf, denbuf,
              acc_s, den_s, sem):
    core = lax.axis_index("c")
    sub = lax.axis_index("s")
    iota = lax.iota(_i32, LN)

    def zdb(i, _):
        w = i * LN + iota
        plsc.store_scatter(denbuf, [w // 8, w % 8], jnp.zeros((LN,), _f32))
        return 0

    lax.fori_loop(0, BB * 8 // LN, zdb, 0)
    for k in range(NROW // NRC):
        pltpu.sync_copy(denbuf.at[pl.ds(0, NRC)],
                        den_s.at[pl.ds(sub * NROW + k * NRC, NRC)])
    pltpu.sync_copy(ad_hbm.at[core], adt_l)
    ov1 = jnp.where(iota >= 8, 1, 0)
    om2 = iota % 2
    hsel = iota // 2

    def half(hh, _):
        _zero_buf(msgbuf, C1W)
        _zero_acc(msgbuf, acc_s, sub)
        plsc.subcore_barrier()

        def chunk(ci, _):
            g = sub * 2 + ci
            pltpu.sync_copy(src_hbm.at[pl.ds(g * CHUNK, CHUNK)], src_l)
            pltpu.sync_copy(dst_hbm.at[pl.ds(g * CHUNK, CHUNK)], dst_l)
            base = g * CHUNK

            def mkidx(i, _):
                v = src_l[pl.ds(i * LN, LN)]
                idx2_l[pl.ds(i * LN, LN)] = v * 4 + core * 2 + hh
                return 0

            lax.fori_loop(0, CHUNK // LN, mkidx, 0)

            def batch(b, _):
                pltpu.async_copy(t1_hbm.at[idx2_l.at[pl.ds(b * BB, BB)]],
                                 rowbuf, sem).wait()

                def group(g2, _):
                    dstv = dst_l[pl.ds(b * BB + g2 * LN, LN)]
                    rowi = g2 * LN + iota
                    valid = (base + b * BB + rowi) < ET
                    for h in range(2):
                        hf = jnp.full((LN,), h, _i32)
                        a_s = plsc.load_gather(rowbuf, [rowi, hf + 16])
                        a_d = plsc.load_gather(adt_l,
                                               [dstv, hf + hh * 2])
                        al = a_s + a_d
                        al = jnp.where(al > 0.0, al, 0.2 * al)
                        p = jnp.where(valid, jnp.exp(al), 0.0)
                        plsc.store_scatter(pbuf, [iota, hf], p)
                    for e in range(16):
                        er = g2 * LN + e
                        ef = jnp.full((LN,), e, _i32)
                        pe = plsc.load_gather(pbuf, [ef, ov1])
                        msgbuf[er, pl.ds(0, LN)] = (
                            rowbuf[er, pl.ds(0, LN)] * pe)
                        pt = plsc.load_gather(pbuf, [ef, om2])
                        pt = jnp.where(hsel == hh, pt, 0.0)
                        plsc.store_scatter(
                            denbuf, [jnp.full((LN,), er, _i32), iota],
                            pt, mask=iota < 8)
                    return 0

                lax.fori_loop(0, BB // LN, group, 0)
                dsl = dst_l.at[pl.ds(b * BB, BB)]
                pltpu.sync_copy(msgbuf, acc_s.at[dsl], add=True)
                pltpu.sync_copy(denbuf, den_s.at[dsl], add=True)
                return 0

            lax.fori_loop(0, NBATCH, batch, 0)
            return 0

        lax.fori_loop(0, 2, chunk, 0)
        plsc.subcore_barrier()
        _copy_out(msgbuf, acc_s, out_hbm, core * 2 + hh, sub)
        plsc.subcore_barrier()
        return 0

    lax.fori_loop(0, 2, half, 0)
    for k in range(NROW // NRC):
        rows = pl.ds(sub * NROW + k * NRC, NRC)
        pltpu.sync_copy(den_s.at[rows], denbuf.at[pl.ds(0, NRC)])
        pltpu.sync_copy(denbuf.at[pl.ds(0, NRC)], dn_hbm.at[core, rows])


def _sc_phase1(t1pk, adt, srcg, dstg):
    mesh = plsc.VectorSubcoreMesh(
        core_axis_name="c", subcore_axis_name="s",
        num_cores=NC, num_subcores=NS)
    f = functools.partial(
        pl.kernel,
        out_type=[
            jax.ShapeDtypeStruct((4, NN, C1W), _f32),
            jax.ShapeDtypeStruct((NC, NN, 8), _f32),
        ],
        mesh=mesh,
        compiler_params=pltpu.CompilerParams(
            needs_layout_passes=False, use_tc_tiling_on_sc=False),
        scratch_types=[
            pltpu.VMEM((NN, 4), _f32),
            pltpu.VMEM((CHUNK,), _i32),
            pltpu.VMEM((CHUNK,), _i32),
            pltpu.VMEM((CHUNK,), _i32),
            pltpu.VMEM((BB, T1W), _f32),
            pltpu.VMEM((BB, C1W), _f32),
            pltpu.VMEM((LN, 2), _f32),
            pltpu.VMEM((BB, 8), _f32),
            pltpu.VMEM_SHARED((NN, C1W), _f32),
            pltpu.VMEM_SHARED((NN, 8), _f32),
            pltpu.SemaphoreType.DMA,
        ],
    )(_sc1_body)
    return f(t1pk, adt, srcg, dstg)


def _sc2_body(h2_hbm, as2_hbm, ad2_hbm, src_hbm, dst_hbm, out_hbm, dn_hbm,
              as2_l, ad2_l, src_l, idx2_l, dst_l, rowbuf, msgbuf, qbuf,
              den_l, idr_l, acc_s, den_s, sem):
    core = lax.axis_index("c")
    sub = lax.axis_index("s")
    iota = lax.iota(_i32, LN)

    def zden(i, _):
        den_l[pl.ds(i * LN, LN)] = jnp.zeros((LN,), _f32)
        return 0

    lax.fori_loop(0, NN // LN, zden, 0)

    @pl.when(sub == 0)
    def _():
        pltpu.sync_copy(den_l, den_s)

    pltpu.sync_copy(as2_hbm, as2_l)
    pltpu.sync_copy(ad2_hbm, ad2_l)

    def half(hh, _):
        _zero_buf(msgbuf, C2W)
        _zero_acc(msgbuf, acc_s, sub)
        plsc.subcore_barrier()

        def chunk(ci, _):
            g = sub * 2 + ci
            pltpu.sync_copy(src_hbm.at[pl.ds(g * CHUNK, CHUNK)], src_l)
            pltpu.sync_copy(dst_hbm.at[pl.ds(g * CHUNK, CHUNK)], dst_l)
            base = g * CHUNK

            def mkidx(i, _):
                v = src_l[pl.ds(i * LN, LN)]
                idx2_l[pl.ds(i * LN, LN)] = v * 8 + core * 4 + hh
                return 0

            lax.fori_loop(0, CHUNK // LN, mkidx, 0)

            def batch(b, _):
                pltpu.async_copy(h2_hbm.at[idx2_l.at[pl.ds(b * BB, BB)]],
                                 rowbuf, sem).wait()

                def group(g2, _):
                    srcv = src_l[pl.ds(b * BB + g2 * LN, LN)]
                    dstv = dst_l[pl.ds(b * BB + g2 * LN, LN)]
                    rowi = g2 * LN + iota
                    valid = (base + b * BB + rowi) < ET
                    a_s = plsc.load_gather(as2_l, [srcv])
                    a_d = plsc.load_gather(ad2_l, [dstv])
                    al = a_s + a_d
                    al = jnp.where(al > 0.0, al, 0.2 * al)
                    q = jnp.where(valid, jnp.exp(al), 0.0)
                    qbuf[...] = q
                    for e in range(16):
                        er = g2 * LN + e
                        qe = plsc.load_gather(
                            qbuf, [jnp.full((LN,), e, _i32)])
                        for j in range(C2W // LN):
                            msgbuf[er, pl.ds(j * LN, LN)] = (
                                rowbuf[er, pl.ds(j * LN, LN)] * qe)
                        plsc.addupdate_scatter(den_l, [dstv], q,
                                               mask=iota == e)
                    return 0

                lax.fori_loop(0, BB // LN, group, 0)
                pltpu.sync_copy(msgbuf,
                                acc_s.at[dst_l.at[pl.ds(b * BB, BB)]],
                                add=True)
                return 0

            lax.fori_loop(0, NBATCH, batch, 0)
            return 0

        lax.fori_loop(0, 2, chunk, 0)
        plsc.subcore_barrier()
        _copy_out(msgbuf, acc_s, out_hbm, core * 4 + hh, sub)
        plsc.subcore_barrier()
        return 0

    lax.fori_loop(0, 4, half, 0)

    def dred(k, _):
        rbase = k * BB
        for t in range(BB // LN):
            idr_l[pl.ds(t * LN, LN)] = rbase + t * LN + iota
        pltpu.sync_copy(den_l.at[pl.ds(rbase, BB)], den_s.at[idr_l],
                        add=True)
        return 0

    lax.fori_loop(0, NN // BB, dred, 0)
    plsc.subcore_barrier()

    @pl.when(sub == 0)
    def _():
        pltpu.sync_copy(den_s, den_l)
        pltpu.sync_copy(den_l, dn_hbm.at[core])


def _sc_phase2(h2pk, as2, ad2, srcg, dstg):
    mesh = plsc.VectorSubcoreMesh(
        core_axis_name="c", subcore_axis_name="s",
        num_cores=NC, num_subcores=NS)
    f = functools.partial(
        pl.kernel,
        out_type=[
            jax.ShapeDtypeStruct((8, NN, C2W), _f32),
            jax.ShapeDtypeStruct((NC, NN), _f32),
        ],
        mesh=mesh,
        compiler_params=pltpu.CompilerParams(
            needs_layout_passes=False, use_tc_tiling_on_sc=False),
        scratch_types=[
            pltpu.VMEM((NN,), _f32),
            pltpu.VMEM((NN,), _f32),
            pltpu.VMEM((CHUNK,), _i32),
            pltpu.VMEM((CHUNK,), _i32),
            pltpu.VMEM((CHUNK,), _i32),
            pltpu.VMEM((BB, C2W), _f32),
            pltpu.VMEM((BB, C2W), _f32),
            pltpu.VMEM((LN,), _f32),
            pltpu.VMEM((NN,), _f32),
            pltpu.VMEM((BB,), _i32),
            pltpu.VMEM_SHARED((NN, C2W), _f32),
            pltpu.VMEM_SHARED((NN,), _f32),
            pltpu.SemaphoreType.DMA,
        ],
    )(_sc2_body)
    return f(h2pk, as2, ad2, srcg, dstg)


@jax.jit
def kernel(x, edge_index, W1, a_src1, a_dst1, b1, W2, a_src2, a_dst2, b2):
    loop = jnp.arange(NN, dtype=_i32)
    src = jnp.concatenate([edge_index[0].astype(_i32), loop])
    dst = jnp.concatenate([edge_index[1].astype(_i32), loop])
    pad = jnp.zeros((EP - ET,), _i32)
    srcg = jnp.concatenate([src, pad])
    dstg = jnp.concatenate([dst, pad])

    diag = jnp.repeat(jnp.eye(8, dtype=_f32), 8, axis=0)
    ms = diag * a_src1.reshape(64)[:, None]
    md = diag * a_dst1.reshape(64)[:, None]
    rexp = jnp.repeat(jnp.eye(8, dtype=_f32), 8, axis=1)
    m2 = jnp.stack([a_src2.reshape(256), a_dst2.reshape(256)], axis=1)

    t1, adt = _tc_a(x, W1, ms, md)
    t1pk = t1.reshape(4 * NN, T1W)
    acc1, dn1 = _sc_phase1(t1pk, adt, srcg, dstg)
    h2, aa = _tc_b(acc1, dn1, b1.reshape(1, 64), W2, m2, rexp)
    h2pk = h2.reshape(8 * NN, C2W)
    acc2, den2 = _sc_phase2(h2pk, aa[:, 0], aa[:, 1], srcg, dstg)
    return _tc_c(acc2, den2[0].reshape(NN, 1),
                 den2[1].reshape(NN, 1), b2.reshape(1, 256))


# speedup vs baseline: 22.9097x; 1.3537x over previous
import functools

import jax
import jax.numpy as jnp
from jax import lax
from jax.experimental import pallas as pl
from jax.experimental.pallas import tpu as pltpu
from jax.experimental.pallas import tpu_sc as plsc

NN = 10000
ET = 160000 + NN
NC, NS, LN = 2, 16, 16
NWORK = NC * NS
BB = 128
NBATCH = 42
CHUNK = BB * NBATCH
EP = CHUNK * NWORK
T1W = 32
C1W = 16
C2W = 32
NROW = NN // NS
NRC = 125

_i32 = jnp.int32
_f32 = jnp.float32


def _tc_a_body(x_ref, w1_ref, ms_ref, md_ref, t1_ref, ad_ref):
    h1 = jnp.dot(x_ref[...], w1_ref[...], preferred_element_type=_f32)
    as1 = jnp.dot(h1, ms_ref[...], preferred_element_type=_f32)
    ad1 = jnp.dot(h1, md_ref[...], preferred_element_type=_f32)
    blk = h1.shape[0]
    zpad = jnp.zeros((blk, T1W - 18), _f32)
    for k in range(4):
        t1_ref[:, k, :] = jnp.concatenate(
            [h1[:, k * 16:(k + 1) * 16], as1[:, k * 2:k * 2 + 2], zpad],
            axis=1)
    for c in range(NC):
        ad_ref[c, :, :] = ad1[:, c * 4:(c + 1) * 4]


def _tc_a(x, W1, ms, md):
    blk = 1000
    return pl.pallas_call(
        _tc_a_body,
        grid=(NN // blk,),
        in_specs=[
            pl.BlockSpec((blk, 256), lambda i: (i, 0)),
            pl.BlockSpec((256, 64), lambda i: (0, 0)),
            pl.BlockSpec((64, 8), lambda i: (0, 0)),
            pl.BlockSpec((64, 8), lambda i: (0, 0)),
        ],
        out_specs=[
            pl.BlockSpec((blk, 4, T1W), lambda i: (i, 0, 0)),
            pl.BlockSpec((NC, blk, 4), lambda i: (0, i, 0)),
        ],
        out_shape=[
            jax.ShapeDtypeStruct((NN, 4, T1W), _f32),
            jax.ShapeDtypeStruct((NC, NN, 4), _f32),
        ],
    )(x, W1, ms, md)


def _tc_b_body(acc_ref, dn_ref, b1_ref, w2_ref, m2_ref, r_ref,
               h2_ref, aa_ref):
    msg = jnp.concatenate([acc_ref[k] for k in range(4)], axis=1)
    den = jnp.concatenate([dn_ref[0, :, 0:4], dn_ref[1, :, 0:4]], axis=1)
    denx = jnp.dot(den, r_ref[...], preferred_element_type=_f32)
    h = msg / denx + b1_ref[...]
    h = jnp.where(h > 0.0, h, jnp.exp(h) - 1.0)
    h2 = jnp.dot(h, w2_ref[...], preferred_element_type=_f32)
    aa = jnp.dot(h2, m2_ref[...], preferred_element_type=_f32)
    h2_ref[...] = h2
    aa_ref[...] = aa


def _tc_b(acc1, dn1, b1, W2, m2, rexp):
    blk = 1000
    return pl.pallas_call(
        _tc_b_body,
        grid=(NN // blk,),
        in_specs=[
            pl.BlockSpec((4, blk, C1W), lambda i: (0, i, 0)),
            pl.BlockSpec((NC, blk, 8), lambda i: (0, i, 0)),
            pl.BlockSpec((1, 64), lambda i: (0, 0)),
            pl.BlockSpec((64, 256), lambda i: (0, 0)),
            pl.BlockSpec((256, 2), lambda i: (0, 0)),
            pl.BlockSpec((8, 64), lambda i: (0, 0)),
        ],
        out_specs=[
            pl.BlockSpec((blk, 256), lambda i: (i, 0)),
            pl.BlockSpec((blk, 2), lambda i: (i, 0)),
        ],
        out_shape=[
            jax.ShapeDtypeStruct((NN, 256), _f32),
            jax.ShapeDtypeStruct((NN, 2), _f32),
        ],
    )(acc1, dn1, b1, W2, m2, rexp)


def _tc_c_body(acc_ref, da_ref, db_ref, b2_ref, o_ref):
    oa = jnp.concatenate([acc_ref[k] for k in range(4)], axis=1) / (
        da_ref[...] * 0.25)
    ob = jnp.concatenate([acc_ref[k] for k in range(4, 8)], axis=1) / (
        db_ref[...] * 0.25)
    o = jnp.concatenate([oa, ob], axis=1) + b2_ref[...]
    m = jnp.max(o, axis=1, keepdims=True)
    lse = m + jnp.log(jnp.sum(jnp.exp(o - m), axis=1, keepdims=True))
    o_ref[...] = o - lse


def _tc_c(acc2, den_a, den_b, b2):
    blk = 1000
    return pl.pallas_call(
        _tc_c_body,
        grid=(NN // blk,),
        in_specs=[
            pl.BlockSpec((8, blk, C2W), lambda i: (0, i, 0)),
            pl.BlockSpec((blk, 1), lambda i: (i, 0)),
            pl.BlockSpec((blk, 1), lambda i: (i, 0)),
            pl.BlockSpec((1, 256), lambda i: (0, 0)),
        ],
        out_specs=pl.BlockSpec((blk, 256), lambda i: (i, 0)),
        out_shape=jax.ShapeDtypeStruct((NN, 256), _f32),
    )(acc2, den_a, den_b, b2)


def _zero_buf(msgbuf, width):
    offs = list(range(0, width - LN + 1, LN))
    if width % LN:
        offs.append(width - LN)

    def zrow(r, _):
        for o in offs:
            msgbuf[r, pl.ds(o, LN)] = jnp.zeros((LN,), _f32)
        return 0

    lax.fori_loop(0, NRC, zrow, 0)


def _zero_acc(msgbuf, acc_s, sub):
    for k in range(NROW // NRC):
        pltpu.sync_copy(msgbuf.at[pl.ds(0, NRC)],
                        acc_s.at[pl.ds(sub * NROW + k * NRC, NRC)])


def _copy_out(msgbuf, acc_s, out_hbm, core, sub):
    for k in range(NROW // NRC):
        rows = pl.ds(sub * NROW + k * NRC, NRC)
        pltpu.sync_copy(acc_s.at[rows], msgbuf.at[pl.ds(0, NRC)])
        pltpu.sync_copy(msgbuf.at[pl.ds(0, NRC)], out_hbm.at[core, rows])


def _mk_idx2(src_l, idx2_l, core):
    def mkidx(i, _):
        v = src_l[pl.ds(i * LN, LN)]
        idx2_l[pl.ds(i * LN, LN)] = v * 2 + core
        return 0

    lax.fori_loop(0, CHUNK // LN, mkidx, 0)


def _sc1_body(t1_hbm, ad_hbm, src_hbm, dst_hbm, out_hbm, dn_hbm,
              adt_l, src_l, idx2_l, dst_l, rowbuf, rowbuf2, msgbuf, pbuf,
              denbuf, acc_s, den_s, sem, dsem, gsem0, gsem1):
    gsems = [gsem0, gsem1]
    core = lax.axis_index("c")
    sub = lax.axis_index("s")
    iota = lax.iota(_i32, LN)

    def zdb(i, _):
        w = i * LN + iota
        plsc.store_scatter(denbuf, [w // 8, w % 8], jnp.zeros((LN,), _f32))
        return 0

    lax.fori_loop(0, BB * 8 // LN, zdb, 0)
    for k in range(NROW // NRC):
        pltpu.sync_copy(denbuf.at[pl.ds(0, NRC)],
                        den_s.at[pl.ds(sub * NROW + k * NRC, NRC)])
    pltpu.sync_copy(ad_hbm.at[core], adt_l)
    ov1 = jnp.where(iota >= 8, 1, 0)
    om2 = iota % 2
    hsel = iota // 2

    def half(hh, _):
        _zero_buf(msgbuf, C1W)
        _zero_acc(msgbuf, acc_s, sub)
        plsc.subcore_barrier()

        def chunk(ci, _):
            g = sub * 2 + ci
            pltpu.sync_copy(src_hbm.at[pl.ds(g * CHUNK, CHUNK)], src_l)
            pltpu.sync_copy(dst_hbm.at[pl.ds(g * CHUNK, CHUNK)], dst_l)
            base = g * CHUNK

            def mkidx(i, _):
                v = src_l[pl.ds(i * LN, LN)]
                idx2_l[pl.ds(i * LN, LN)] = v * 4 + core * 2 + hh
                return 0

            lax.fori_loop(0, CHUNK // LN, mkidx, 0)

            rbufs = [rowbuf, rowbuf2]
            pltpu.async_copy(t1_hbm.at[idx2_l.at[pl.ds(0, BB)]],
                             rbufs[0], gsems[0])

            def run_batch(b, i, k, first):
                rb = rbufs[k]
                dsl = dst_l.at[pl.ds(b * BB, BB)]

                @pl.when(jnp.logical_not(first))
                def _():
                    pltpu.make_async_copy(msgbuf, acc_s.at[dsl],
                                          sem).wait()
                    pltpu.make_async_copy(denbuf, den_s.at[dsl],
                                          dsem).wait()

                pltpu.make_async_copy(
                    t1_hbm.at[idx2_l.at[pl.ds(b * BB, BB)]], rb,
                    gsems[k]).wait()

                @pl.when(b + 1 < NBATCH)
                def _():
                    pltpu.async_copy(
                        t1_hbm.at[idx2_l.at[pl.ds((b + 1) * BB, BB)]],
                        rbufs[1 - k], gsems[1 - k])

                def group(g2, _):
                    dstv = dst_l[pl.ds(b * BB + g2 * LN, LN)]
                    rowi = g2 * LN + iota
                    valid = (base + b * BB + rowi) < ET
                    for h in range(2):
                        hf = jnp.full((LN,), h, _i32)
                        a_s = plsc.load_gather(rb, [rowi, hf + 16])
                        a_d = plsc.load_gather(adt_l,
                                               [dstv, hf + hh * 2])
                        al = a_s + a_d
                        al = jnp.where(al > 0.0, al, 0.2 * al)
                        p = jnp.where(valid, jnp.exp(al), 0.0)
                        plsc.store_scatter(pbuf, [iota, hf], p)
                    for e in range(16):
                        er = g2 * LN + e
                        ef = jnp.full((LN,), e, _i32)
                        pe = plsc.load_gather(pbuf, [ef, ov1])
                        msgbuf[er, pl.ds(0, LN)] = (
                            rb[er, pl.ds(0, LN)] * pe)
                        pt = plsc.load_gather(pbuf, [ef, om2])
                        pt = jnp.where(hsel == hh, pt, 0.0)
                        plsc.store_scatter(
                            denbuf, [jnp.full((LN,), er, _i32), iota],
                            pt, mask=iota < 8)
                    return 0

                lax.fori_loop(0, BB // LN, group, 0)
                pltpu.async_copy(msgbuf, acc_s.at[dsl], sem, add=True)
                pltpu.async_copy(denbuf, den_s.at[dsl], dsem, add=True)

            def batch2(i, _):
                run_batch(i * 2, i, 0, i == 0)
                run_batch(i * 2 + 1, i, 1, False)
                return 0

            lax.fori_loop(0, NBATCH // 2, batch2, 0)
            dsl = dst_l.at[pl.ds((NBATCH - 1) * BB, BB)]
            pltpu.make_async_copy(msgbuf, acc_s.at[dsl], sem).wait()
            pltpu.make_async_copy(denbuf, den_s.at[dsl], dsem).wait()
            return 0

        lax.fori_loop(0, 2, chunk, 0)
        plsc.subcore_barrier()
        _copy_out(msgbuf, acc_s, out_hbm, core * 2 + hh, sub)
        plsc.subcore_barrier()
        return 0

    lax.fori_loop(0, 2, half, 0)
    for k in range(NROW // NRC):
        rows = pl.ds(sub * NROW + k * NRC, NRC)
        pltpu.sync_copy(den_s.at[rows], denbuf.at[pl.ds(0, NRC)])
        pltpu.sync_copy(denbuf.at[pl.ds(0, NRC)], dn_hbm.at[core, rows])


def _sc_phase1(t1pk, adt, srcg, dstg):
    mesh = plsc.VectorSubcoreMesh(
        core_axis_name="c", subcore_axis_name="s",
        num_cores=NC, num_subcores=NS)
    f = functools.partial(
        pl.kernel,
        out_type=[
            jax.ShapeDtypeStruct((4, NN, C1W), _f32),
            jax.ShapeDtypeStruct((NC, NN, 8), _f32),
        ],
        mesh=mesh,
        compiler_params=pltpu.CompilerParams(
            needs_layout_passes=False, use_tc_tiling_on_sc=False),
        scratch_types=[
            pltpu.VMEM((NN, 4), _f32),
            pltpu.VMEM((CHUNK,), _i32),
            pltpu.VMEM((CHUNK,), _i32),
            pltpu.VMEM((CHUNK,), _i32),
            pltpu.VMEM((BB, T1W), _f32),
            pltpu.VMEM((BB, T1W), _f32),
            pltpu.VMEM((BB, C1W), _f32),
            pltpu.VMEM((LN, 2), _f32),
            pltpu.VMEM((BB, 8), _f32),
            pltpu.VMEM_SHARED((NN, C1W), _f32),
            pltpu.VMEM_SHARED((NN, 8), _f32),
            pltpu.SemaphoreType.DMA,
            pltpu.SemaphoreType.DMA,
            pltpu.SemaphoreType.DMA,
            pltpu.SemaphoreType.DMA,
        ],
    )(_sc1_body)
    return f(t1pk, adt, srcg, dstg)


def _sc2_body(h2_hbm, as2_hbm, ad2_hbm, src_hbm, dst_hbm, out_hbm, dn_hbm,
              as2_l, ad2_l, src_l, idx2_l, dst_l, rowbuf, rowbuf2, msgbuf,
              qbuf, den_l, idr_l, acc_s, den_s, sem, gsem0, gsem1):
    gsems = [gsem0, gsem1]
    core = lax.axis_index("c")
    sub = lax.axis_index("s")
    iota = lax.iota(_i32, LN)

    def zden(i, _):
        den_l[pl.ds(i * LN, LN)] = jnp.zeros((LN,), _f32)
        return 0

    lax.fori_loop(0, NN // LN, zden, 0)

    @pl.when(sub == 0)
    def _():
        pltpu.sync_copy(den_l, den_s)

    pltpu.sync_copy(as2_hbm, as2_l)
    pltpu.sync_copy(ad2_hbm, ad2_l)

    def half(hh, _):
        _zero_buf(msgbuf, C2W)
        _zero_acc(msgbuf, acc_s, sub)
        plsc.subcore_barrier()

        def chunk(ci, _):
            g = sub * 2 + ci
            pltpu.sync_copy(src_hbm.at[pl.ds(g * CHUNK, CHUNK)], src_l)
            pltpu.sync_copy(dst_hbm.at[pl.ds(g * CHUNK, CHUNK)], dst_l)
            base = g * CHUNK

            def mkidx(i, _):
                v = src_l[pl.ds(i * LN, LN)]
                idx2_l[pl.ds(i * LN, LN)] = v * 8 + core * 4 + hh
                return 0

            lax.fori_loop(0, CHUNK // LN, mkidx, 0)

            rbufs = [rowbuf, rowbuf2]
            pltpu.async_copy(h2_hbm.at[idx2_l.at[pl.ds(0, BB)]],
                             rbufs[0], gsems[0])

            def run_batch(b, i, k, first):
                rb = rbufs[k]
                dsl = dst_l.at[pl.ds(b * BB, BB)]

                @pl.when(jnp.logical_not(first))
                def _():
                    pltpu.make_async_copy(msgbuf, acc_s.at[dsl],
                                          sem).wait()

                pltpu.make_async_copy(
                    h2_hbm.at[idx2_l.at[pl.ds(b * BB, BB)]], rb,
                    gsems[k]).wait()

                @pl.when(b + 1 < NBATCH)
                def _():
                    pltpu.async_copy(
                        h2_hbm.at[idx2_l.at[pl.ds((b + 1) * BB, BB)]],
                        rbufs[1 - k], gsems[1 - k])

                def group(g2, _):
                    srcv = src_l[pl.ds(b * BB + g2 * LN, LN)]
                    dstv = dst_l[pl.ds(b * BB + g2 * LN, LN)]
                    rowi = g2 * LN + iota
                    valid = (base + b * BB + rowi) < ET
                    a_s = plsc.load_gather(as2_l, [srcv])
                    a_d = plsc.load_gather(ad2_l, [dstv])
                    al = a_s + a_d
                    al = jnp.where(al > 0.0, al, 0.2 * al)
                    q = jnp.where(valid, jnp.exp(al), 0.0)
                    qbuf[...] = q
                    for e in range(16):
                        er = g2 * LN + e
                        qe = plsc.load_gather(
                            qbuf, [jnp.full((LN,), e, _i32)])
                        for j in range(C2W // LN):
                            msgbuf[er, pl.ds(j * LN, LN)] = (
                                rb[er, pl.ds(j * LN, LN)] * qe)
                        plsc.addupdate_scatter(den_l, [dstv], q,
                                               mask=iota == e)
                    return 0

                lax.fori_loop(0, BB // LN, group, 0)
                pltpu.async_copy(msgbuf, acc_s.at[dsl], sem, add=True)

            def batch2(i, _):
                run_batch(i * 2, i, 0, i == 0)
                run_batch(i * 2 + 1, i, 1, False)
                return 0

            lax.fori_loop(0, NBATCH // 2, batch2, 0)
            dsl = dst_l.at[pl.ds((NBATCH - 1) * BB, BB)]
            pltpu.make_async_copy(msgbuf, acc_s.at[dsl], sem).wait()
            return 0

        lax.fori_loop(0, 2, chunk, 0)
        plsc.subcore_barrier()
        _copy_out(msgbuf, acc_s, out_hbm, core * 4 + hh, sub)
        plsc.subcore_barrier()
        return 0

    lax.fori_loop(0, 4, half, 0)

    def dred(k, _):
        rbase = k * BB
        for t in range(BB // LN):
            idr_l[pl.ds(t * LN, LN)] = rbase + t * LN + iota
        pltpu.sync_copy(den_l.at[pl.ds(rbase, BB)], den_s.at[idr_l],
                        add=True)
        return 0

    lax.fori_loop(0, NN // BB, dred, 0)
    plsc.subcore_barrier()

    @pl.when(sub == 0)
    def _():
        pltpu.sync_copy(den_s, den_l)
        pltpu.sync_copy(den_l, dn_hbm.at[core])


def _sc_phase2(h2pk, as2, ad2, srcg, dstg):
    mesh = plsc.VectorSubcoreMesh(
        core_axis_name="c", subcore_axis_name="s",
        num_cores=NC, num_subcores=NS)
    f = functools.partial(
        pl.kernel,
        out_type=[
            jax.ShapeDtypeStruct((8, NN, C2W), _f32),
            jax.ShapeDtypeStruct((NC, NN), _f32),
        ],
        mesh=mesh,
        compiler_params=pltpu.CompilerParams(
            needs_layout_passes=False, use_tc_tiling_on_sc=False),
        scratch_types=[
            pltpu.VMEM((NN,), _f32),
            pltpu.VMEM((NN,), _f32),
            pltpu.VMEM((CHUNK,), _i32),
            pltpu.VMEM((CHUNK,), _i32),
            pltpu.VMEM((CHUNK,), _i32),
            pltpu.VMEM((BB, C2W), _f32),
            pltpu.VMEM((BB, C2W), _f32),
            pltpu.VMEM((BB, C2W), _f32),
            pltpu.VMEM((LN,), _f32),
            pltpu.VMEM((NN,), _f32),
            pltpu.VMEM((BB,), _i32),
            pltpu.VMEM_SHARED((NN, C2W), _f32),
            pltpu.VMEM_SHARED((NN,), _f32),
            pltpu.SemaphoreType.DMA,
            pltpu.SemaphoreType.DMA,
            pltpu.SemaphoreType.DMA,
        ],
    )(_sc2_body)
    return f(h2pk, as2, ad2, srcg, dstg)


@jax.jit
def kernel(x, edge_index, W1, a_src1, a_dst1, b1, W2, a_src2, a_dst2, b2):
    loop = jnp.arange(NN, dtype=_i32)
    src = jnp.concatenate([edge_index[0].astype(_i32), loop])
    dst = jnp.concatenate([edge_index[1].astype(_i32), loop])
    pad = jnp.zeros((EP - ET,), _i32)
    srcg = jnp.concatenate([src, pad])
    dstg = jnp.concatenate([dst, pad])

    diag = jnp.repeat(jnp.eye(8, dtype=_f32), 8, axis=0)
    ms = diag * a_src1.reshape(64)[:, None]
    md = diag * a_dst1.reshape(64)[:, None]
    rexp = jnp.repeat(jnp.eye(8, dtype=_f32), 8, axis=1)
    m2 = jnp.stack([a_src2.reshape(256), a_dst2.reshape(256)], axis=1)

    t1, adt = _tc_a(x, W1, ms, md)
    t1pk = t1.reshape(4 * NN, T1W)
    acc1, dn1 = _sc_phase1(t1pk, adt, srcg, dstg)
    h2, aa = _tc_b(acc1, dn1, b1.reshape(1, 64), W2, m2, rexp)
    h2pk = h2.reshape(8 * NN, C2W)
    acc2, den2 = _sc_phase2(h2pk, aa[:, 0], aa[:, 1], srcg, dstg)
    return _tc_c(acc2, den2[0].reshape(NN, 1),
                 den2[1].reshape(NN, 1), b2.reshape(1, 256))


# speedup vs baseline: 22.9154x; 1.0002x over previous
import functools

import jax
import jax.numpy as jnp
from jax import lax
from jax.experimental import pallas as pl
from jax.experimental.pallas import tpu as pltpu
from jax.experimental.pallas import tpu_sc as plsc

NN = 10000
ET = 160000 + NN
NC, NS, LN = 2, 16, 16
NWORK = NC * NS
BB = 128
NBATCH = 42
CHUNK = BB * NBATCH
EP = CHUNK * NWORK
T1W = 32
C1W = 16
C2W = 32
NROW = NN // NS
NRC = 125

_i32 = jnp.int32
_f32 = jnp.float32


def _tc_a_body(x_ref, w1_ref, ms_ref, md_ref, t1_ref, ad_ref):
    h1 = jnp.dot(x_ref[...], w1_ref[...], preferred_element_type=_f32)
    as1 = jnp.dot(h1, ms_ref[...], preferred_element_type=_f32)
    ad1 = jnp.dot(h1, md_ref[...], preferred_element_type=_f32)
    blk = h1.shape[0]
    zpad = jnp.zeros((blk, T1W - 18), _f32)
    for k in range(4):
        t1_ref[:, k, :] = jnp.concatenate(
            [h1[:, k * 16:(k + 1) * 16], as1[:, k * 2:k * 2 + 2], zpad],
            axis=1)
    for c in range(NC):
        ad_ref[c, :, :] = ad1[:, c * 4:(c + 1) * 4]


def _tc_a(x, W1, ms, md):
    blk = 1000
    return pl.pallas_call(
        _tc_a_body,
        grid=(NN // blk,),
        in_specs=[
            pl.BlockSpec((blk, 256), lambda i: (i, 0)),
            pl.BlockSpec((256, 64), lambda i: (0, 0)),
            pl.BlockSpec((64, 8), lambda i: (0, 0)),
            pl.BlockSpec((64, 8), lambda i: (0, 0)),
        ],
        out_specs=[
            pl.BlockSpec((blk, 4, T1W), lambda i: (i, 0, 0)),
            pl.BlockSpec((NC, blk, 4), lambda i: (0, i, 0)),
        ],
        out_shape=[
            jax.ShapeDtypeStruct((NN, 4, T1W), _f32),
            jax.ShapeDtypeStruct((NC, NN, 4), _f32),
        ],
    )(x, W1, ms, md)


def _tc_b_body(acc_ref, dn_ref, b1_ref, w2_ref, m2_ref, r_ref,
               h2_ref, aa_ref):
    msg = jnp.concatenate([acc_ref[k] for k in range(4)], axis=1)
    den = jnp.concatenate([dn_ref[0, :, 0:4], dn_ref[1, :, 0:4]], axis=1)
    denx = jnp.dot(den, r_ref[...], preferred_element_type=_f32)
    h = msg / denx + b1_ref[...]
    h = jnp.where(h > 0.0, h, jnp.exp(h) - 1.0)
    h2 = jnp.dot(h, w2_ref[...], preferred_element_type=_f32)
    aa = jnp.dot(h2, m2_ref[...], preferred_element_type=_f32)
    h2_ref[...] = h2
    aa_ref[...] = aa


def _tc_b(acc1, dn1, b1, W2, m2, rexp):
    blk = 1000
    return pl.pallas_call(
        _tc_b_body,
        grid=(NN // blk,),
        in_specs=[
            pl.BlockSpec((4, blk, C1W), lambda i: (0, i, 0)),
            pl.BlockSpec((NC, blk, 8), lambda i: (0, i, 0)),
            pl.BlockSpec((1, 64), lambda i: (0, 0)),
            pl.BlockSpec((64, 256), lambda i: (0, 0)),
            pl.BlockSpec((256, 2), lambda i: (0, 0)),
            pl.BlockSpec((8, 64), lambda i: (0, 0)),
        ],
        out_specs=[
            pl.BlockSpec((blk, 256), lambda i: (i, 0)),
            pl.BlockSpec((blk, 2), lambda i: (i, 0)),
        ],
        out_shape=[
            jax.ShapeDtypeStruct((NN, 256), _f32),
            jax.ShapeDtypeStruct((NN, 2), _f32),
        ],
    )(acc1, dn1, b1, W2, m2, rexp)


def _tc_c_body(acc_ref, da_ref, db_ref, b2_ref, o_ref):
    oa = jnp.concatenate([acc_ref[k] for k in range(4)], axis=1) / da_ref[...]
    ob = jnp.concatenate([acc_ref[k] for k in range(4, 8)],
                         axis=1) / db_ref[...]
    o = jnp.concatenate([oa, ob], axis=1) + b2_ref[...]
    m = jnp.max(o, axis=1, keepdims=True)
    lse = m + jnp.log(jnp.sum(jnp.exp(o - m), axis=1, keepdims=True))
    o_ref[...] = o - lse


def _tc_c(acc2, den_a, den_b, b2):
    blk = 1000
    return pl.pallas_call(
        _tc_c_body,
        grid=(NN // blk,),
        in_specs=[
            pl.BlockSpec((8, blk, C2W), lambda i: (0, i, 0)),
            pl.BlockSpec((blk, 1), lambda i: (i, 0)),
            pl.BlockSpec((blk, 1), lambda i: (i, 0)),
            pl.BlockSpec((1, 256), lambda i: (0, 0)),
        ],
        out_specs=pl.BlockSpec((blk, 256), lambda i: (i, 0)),
        out_shape=jax.ShapeDtypeStruct((NN, 256), _f32),
    )(acc2, den_a, den_b, b2)


def _zero_buf(msgbuf, width):
    offs = list(range(0, width - LN + 1, LN))
    if width % LN:
        offs.append(width - LN)

    def zrow(r, _):
        for o in offs:
            msgbuf[r, pl.ds(o, LN)] = jnp.zeros((LN,), _f32)
        return 0

    lax.fori_loop(0, NRC, zrow, 0)


def _zero_acc(msgbuf, acc_s, sub):
    for k in range(NROW // NRC):
        pltpu.sync_copy(msgbuf.at[pl.ds(0, NRC)],
                        acc_s.at[pl.ds(sub * NROW + k * NRC, NRC)])


def _copy_out(msgbuf, acc_s, out_hbm, core, sub):
    for k in range(NROW // NRC):
        rows = pl.ds(sub * NROW + k * NRC, NRC)
        pltpu.sync_copy(acc_s.at[rows], msgbuf.at[pl.ds(0, NRC)])
        pltpu.sync_copy(msgbuf.at[pl.ds(0, NRC)], out_hbm.at[core, rows])


def _mk_idx2(src_l, idx2_l, core):
    def mkidx(i, _):
        v = src_l[pl.ds(i * LN, LN)]
        idx2_l[pl.ds(i * LN, LN)] = v * 2 + core
        return 0

    lax.fori_loop(0, CHUNK // LN, mkidx, 0)


def _sc1_body(t1_hbm, ad_hbm, src_hbm, dst_hbm, out_hbm, dn_hbm,
              adt_l, src_l, idx2_l, dst_l, rowbuf, rowbuf2, msgbuf, pbuf,
              denbuf, acc_s, den_s, sem, dsem, gsem0, gsem1):
    gsems = [gsem0, gsem1]
    core = lax.axis_index("c")
    sub = lax.axis_index("s")
    iota = lax.iota(_i32, LN)

    def zdb(i, _):
        w = i * LN + iota
        plsc.store_scatter(denbuf, [w // 8, w % 8], jnp.zeros((LN,), _f32))
        return 0

    lax.fori_loop(0, BB * 8 // LN, zdb, 0)
    for k in range(NROW // NRC):
        pltpu.sync_copy(denbuf.at[pl.ds(0, NRC)],
                        den_s.at[pl.ds(sub * NROW + k * NRC, NRC)])
    pltpu.sync_copy(ad_hbm.at[core], adt_l)
    ov1 = jnp.where(iota >= 8, 1, 0)
    om2 = iota % 2
    hsel = iota // 2

    def half(hh, _):
        _zero_buf(msgbuf, C1W)
        _zero_acc(msgbuf, acc_s, sub)
        plsc.subcore_barrier()

        def chunk(ci, _):
            g = sub * 2 + ci
            pltpu.sync_copy(src_hbm.at[pl.ds(g * CHUNK, CHUNK)], src_l)
            pltpu.sync_copy(dst_hbm.at[pl.ds(g * CHUNK, CHUNK)], dst_l)
            base = g * CHUNK

            def mkidx(i, _):
                v = src_l[pl.ds(i * LN, LN)]
                idx2_l[pl.ds(i * LN, LN)] = v * 4 + core * 2 + hh
                return 0

            lax.fori_loop(0, CHUNK // LN, mkidx, 0)

            rbufs = [rowbuf, rowbuf2]
            pltpu.async_copy(t1_hbm.at[idx2_l.at[pl.ds(0, BB)]],
                             rbufs[0], gsems[0])

            def run_batch(b, i, k, first):
                rb = rbufs[k]
                dsl = dst_l.at[pl.ds(b * BB, BB)]

                @pl.when(jnp.logical_not(first))
                def _():
                    pltpu.make_async_copy(msgbuf, acc_s.at[dsl],
                                          sem).wait()
                    pltpu.make_async_copy(denbuf, den_s.at[dsl],
                                          dsem).wait()

                pltpu.make_async_copy(
                    t1_hbm.at[idx2_l.at[pl.ds(b * BB, BB)]], rb,
                    gsems[k]).wait()

                @pl.when(b + 1 < NBATCH)
                def _():
                    pltpu.async_copy(
                        t1_hbm.at[idx2_l.at[pl.ds((b + 1) * BB, BB)]],
                        rbufs[1 - k], gsems[1 - k])

                def group(g2, _):
                    dstv = dst_l[pl.ds(b * BB + g2 * LN, LN)]
                    rowi = g2 * LN + iota
                    valid = (base + b * BB + rowi) < ET
                    for h in range(2):
                        hf = jnp.full((LN,), h, _i32)
                        a_s = plsc.load_gather(rb, [rowi, hf + 16])
                        a_d = plsc.load_gather(adt_l,
                                               [dstv, hf + hh * 2])
                        al = a_s + a_d
                        al = jnp.where(al > 0.0, al, 0.2 * al)
                        p = jnp.where(valid, jnp.exp(al), 0.0)
                        plsc.store_scatter(pbuf, [iota, hf], p)
                    for e in range(16):
                        er = g2 * LN + e
                        ef = jnp.full((LN,), e, _i32)
                        pe = plsc.load_gather(pbuf, [ef, ov1])
                        msgbuf[er, pl.ds(0, LN)] = (
                            rb[er, pl.ds(0, LN)] * pe)
                        pt = plsc.load_gather(pbuf, [ef, om2])
                        pt = jnp.where(hsel == hh, pt, 0.0)
                        plsc.store_scatter(
                            denbuf, [jnp.full((LN,), er, _i32), iota],
                            pt, mask=iota < 8)
                    return 0

                lax.fori_loop(0, BB // LN, group, 0)
                pltpu.async_copy(msgbuf, acc_s.at[dsl], sem, add=True)
                pltpu.async_copy(denbuf, den_s.at[dsl], dsem, add=True)

            def batch2(i, _):
                run_batch(i * 2, i, 0, i == 0)
                run_batch(i * 2 + 1, i, 1, False)
                return 0

            lax.fori_loop(0, NBATCH // 2, batch2, 0)
            dsl = dst_l.at[pl.ds((NBATCH - 1) * BB, BB)]
            pltpu.make_async_copy(msgbuf, acc_s.at[dsl], sem).wait()
            pltpu.make_async_copy(denbuf, den_s.at[dsl], dsem).wait()
            return 0

        lax.fori_loop(0, 2, chunk, 0)
        plsc.subcore_barrier()
        _copy_out(msgbuf, acc_s, out_hbm, core * 2 + hh, sub)
        plsc.subcore_barrier()
        return 0

    lax.fori_loop(0, 2, half, 0)
    for k in range(NROW // NRC):
        rows = pl.ds(sub * NROW + k * NRC, NRC)
        pltpu.sync_copy(den_s.at[rows], denbuf.at[pl.ds(0, NRC)])
        pltpu.sync_copy(denbuf.at[pl.ds(0, NRC)], dn_hbm.at[core, rows])


def _sc_phase1(t1pk, adt, srcg, dstg):
    mesh = plsc.VectorSubcoreMesh(
        core_axis_name="c", subcore_axis_name="s",
        num_cores=NC, num_subcores=NS)
    f = functools.partial(
        pl.kernel,
        out_type=[
            jax.ShapeDtypeStruct((4, NN, C1W), _f32),
            jax.ShapeDtypeStruct((NC, NN, 8), _f32),
        ],
        mesh=mesh,
        compiler_params=pltpu.CompilerParams(
            needs_layout_passes=False, use_tc_tiling_on_sc=False),
        scratch_types=[
            pltpu.VMEM((NN, 4), _f32),
            pltpu.VMEM((CHUNK,), _i32),
            pltpu.VMEM((CHUNK,), _i32),
            pltpu.VMEM((CHUNK,), _i32),
            pltpu.VMEM((BB, T1W), _f32),
            pltpu.VMEM((BB, T1W), _f32),
            pltpu.VMEM((BB, C1W), _f32),
            pltpu.VMEM((LN, 2), _f32),
            pltpu.VMEM((BB, 8), _f32),
            pltpu.VMEM_SHARED((NN, C1W), _f32),
            pltpu.VMEM_SHARED((NN, 8), _f32),
            pltpu.SemaphoreType.DMA,
            pltpu.SemaphoreType.DMA,
            pltpu.SemaphoreType.DMA,
            pltpu.SemaphoreType.DMA,
        ],
    )(_sc1_body)
    return f(t1pk, adt, srcg, dstg)


def _sc2_body(h2_hbm, as2_hbm, ad2_hbm, src_hbm, dst_hbm, out_hbm, dn_hbm,
              as2_l, ad2_l, src_l, idx2_l, dst_l, rowbuf, rowbuf2, msgbuf,
              qbuf, den_l, idr_l, acc_s, den_s, sem, gsem0, gsem1):
    gsems = [gsem0, gsem1]
    core = lax.axis_index("c")
    sub = lax.axis_index("s")
    iota = lax.iota(_i32, LN)

    def zden(i, _):
        den_l[pl.ds(i * LN, LN)] = jnp.zeros((LN,), _f32)
        return 0

    lax.fori_loop(0, NN // LN, zden, 0)

    @pl.when(sub == 0)
    def _():
        pltpu.sync_copy(den_l, den_s)

    pltpu.sync_copy(as2_hbm, as2_l)
    pltpu.sync_copy(ad2_hbm, ad2_l)

    def half(hh, _):
        _zero_buf(msgbuf, C2W)
        _zero_acc(msgbuf, acc_s, sub)
        plsc.subcore_barrier()

        def chunk(ci, _):
            g = sub * 2 + ci
            pltpu.sync_copy(src_hbm.at[pl.ds(g * CHUNK, CHUNK)], src_l)
            pltpu.sync_copy(dst_hbm.at[pl.ds(g * CHUNK, CHUNK)], dst_l)
            base = g * CHUNK

            def mkidx(i, _):
                v = src_l[pl.ds(i * LN, LN)]
                idx2_l[pl.ds(i * LN, LN)] = v * 8 + core * 4 + hh
                return 0

            lax.fori_loop(0, CHUNK // LN, mkidx, 0)

            rbufs = [rowbuf, rowbuf2]
            pltpu.async_copy(h2_hbm.at[idx2_l.at[pl.ds(0, BB)]],
                             rbufs[0], gsems[0])

            def run_batch(b, i, k, first):
                rb = rbufs[k]
                dsl = dst_l.at[pl.ds(b * BB, BB)]

                @pl.when(jnp.logical_not(first))
                def _():
                    pltpu.make_async_copy(msgbuf, acc_s.at[dsl],
                                          sem).wait()

                pltpu.make_async_copy(
                    h2_hbm.at[idx2_l.at[pl.ds(b * BB, BB)]], rb,
                    gsems[k]).wait()

                @pl.when(b + 1 < NBATCH)
                def _():
                    pltpu.async_copy(
                        h2_hbm.at[idx2_l.at[pl.ds((b + 1) * BB, BB)]],
                        rbufs[1 - k], gsems[1 - k])

                def group(g2, _):
                    srcv = src_l[pl.ds(b * BB + g2 * LN, LN)]
                    dstv = dst_l[pl.ds(b * BB + g2 * LN, LN)]
                    rowi = g2 * LN + iota
                    valid = (base + b * BB + rowi) < ET
                    a_s = plsc.load_gather(as2_l, [srcv])
                    a_d = plsc.load_gather(ad2_l, [dstv])
                    al = a_s + a_d
                    al = jnp.where(al > 0.0, al, 0.2 * al)
                    q = jnp.where(valid, jnp.exp(al), 0.0)
                    qbuf[...] = q
                    for e in range(16):
                        er = g2 * LN + e
                        qe = plsc.load_gather(
                            qbuf, [jnp.full((LN,), e, _i32)])
                        for j in range(C2W // LN):
                            msgbuf[er, pl.ds(j * LN, LN)] = (
                                rb[er, pl.ds(j * LN, LN)] * qe)

                    @pl.when(hh == 0)
                    def _():
                        for e in range(16):
                            plsc.addupdate_scatter(den_l, [dstv], q,
                                                   mask=iota == e)

                    return 0

                lax.fori_loop(0, BB // LN, group, 0)
                pltpu.async_copy(msgbuf, acc_s.at[dsl], sem, add=True)

            def batch2(i, _):
                run_batch(i * 2, i, 0, i == 0)
                run_batch(i * 2 + 1, i, 1, False)
                return 0

            lax.fori_loop(0, NBATCH // 2, batch2, 0)
            dsl = dst_l.at[pl.ds((NBATCH - 1) * BB, BB)]
            pltpu.make_async_copy(msgbuf, acc_s.at[dsl], sem).wait()
            return 0

        lax.fori_loop(0, 2, chunk, 0)
        plsc.subcore_barrier()
        _copy_out(msgbuf, acc_s, out_hbm, core * 4 + hh, sub)
        plsc.subcore_barrier()
        return 0

    lax.fori_loop(0, 4, half, 0)

    def dred(k, _):
        rbase = k * BB
        for t in range(BB // LN):
            idr_l[pl.ds(t * LN, LN)] = rbase + t * LN + iota
        pltpu.sync_copy(den_l.at[pl.ds(rbase, BB)], den_s.at[idr_l],
                        add=True)
        return 0

    lax.fori_loop(0, NN // BB, dred, 0)
    plsc.subcore_barrier()

    @pl.when(sub == 0)
    def _():
        pltpu.sync_copy(den_s, den_l)
        pltpu.sync_copy(den_l, dn_hbm.at[core])


def _sc_phase2(h2pk, as2, ad2, srcg, dstg):
    mesh = plsc.VectorSubcoreMesh(
        core_axis_name="c", subcore_axis_name="s",
        num_cores=NC, num_subcores=NS)
    f = functools.partial(
        pl.kernel,
        out_type=[
            jax.ShapeDtypeStruct((8, NN, C2W), _f32),
            jax.ShapeDtypeStruct((NC, NN), _f32),
        ],
        mesh=mesh,
        compiler_params=pltpu.CompilerParams(
            needs_layout_passes=False, use_tc_tiling_on_sc=False),
        scratch_types=[
            pltpu.VMEM((NN,), _f32),
            pltpu.VMEM((NN,), _f32),
            pltpu.VMEM((CHUNK,), _i32),
            pltpu.VMEM((CHUNK,), _i32),
            pltpu.VMEM((CHUNK,), _i32),
            pltpu.VMEM((BB, C2W), _f32),
            pltpu.VMEM((BB, C2W), _f32),
            pltpu.VMEM((BB, C2W), _f32),
            pltpu.VMEM((LN,), _f32),
            pltpu.VMEM((NN,), _f32),
            pltpu.VMEM((BB,), _i32),
            pltpu.VMEM_SHARED((NN, C2W), _f32),
            pltpu.VMEM_SHARED((NN,), _f32),
            pltpu.SemaphoreType.DMA,
            pltpu.SemaphoreType.DMA,
            pltpu.SemaphoreType.DMA,
        ],
    )(_sc2_body)
    return f(h2pk, as2, ad2, srcg, dstg)


@jax.jit
def kernel(x, edge_index, W1, a_src1, a_dst1, b1, W2, a_src2, a_dst2, b2):
    loop = jnp.arange(NN, dtype=_i32)
    src = jnp.concatenate([edge_index[0].astype(_i32), loop])
    dst = jnp.concatenate([edge_index[1].astype(_i32), loop])
    pad = jnp.zeros((EP - ET,), _i32)
    srcg = jnp.concatenate([src, pad])
    dstg = jnp.concatenate([dst, pad])

    diag = jnp.repeat(jnp.eye(8, dtype=_f32), 8, axis=0)
    ms = diag * a_src1.reshape(64)[:, None]
    md = diag * a_dst1.reshape(64)[:, None]
    rexp = jnp.repeat(jnp.eye(8, dtype=_f32), 8, axis=1)
    m2 = jnp.stack([a_src2.reshape(256), a_dst2.reshape(256)], axis=1)

    t1, adt = _tc_a(x, W1, ms, md)
    t1pk = t1.reshape(4 * NN, T1W)
    acc1, dn1 = _sc_phase1(t1pk, adt, srcg, dstg)
    h2, aa = _tc_b(acc1, dn1, b1.reshape(1, 64), W2, m2, rexp)
    h2pk = h2.reshape(8 * NN, C2W)
    acc2, den2 = _sc_phase2(h2pk, aa[:, 0], aa[:, 1], srcg, dstg)
    return _tc_c(acc2, den2[0].reshape(NN, 1),
                 den2[1].reshape(NN, 1), b2.reshape(1, 256))


# speedup vs baseline: 24.7673x; 1.0808x over previous
import functools

import jax
import jax.numpy as jnp
from jax import lax
from jax.experimental import pallas as pl
from jax.experimental.pallas import tpu as pltpu
from jax.experimental.pallas import tpu_sc as plsc

NN = 10000
ET = 160000 + NN
NC, NS, LN = 2, 16, 16
NWORK = NC * NS
BB = 128
NBATCH = 42
CHUNK = BB * NBATCH
EP = CHUNK * NWORK
T1W = 32
C1W = 16
C2W = 32
NROW = NN // NS
NRC = 125

_i32 = jnp.int32
_f32 = jnp.float32


def _tc_a_body(x_ref, w1_ref, ms_ref, md_ref, t1_ref, ad_ref):
    h1 = jnp.dot(x_ref[...], w1_ref[...], preferred_element_type=_f32)
    as1 = jnp.dot(h1, ms_ref[...], preferred_element_type=_f32)
    ad1 = jnp.dot(h1, md_ref[...], preferred_element_type=_f32)
    blk = h1.shape[0]
    zpad = jnp.zeros((blk, T1W - 18), _f32)
    for k in range(4):
        t1_ref[:, k, :] = jnp.concatenate(
            [h1[:, k * 16:(k + 1) * 16], as1[:, k * 2:k * 2 + 2], zpad],
            axis=1)
    for c in range(NC):
        ad_ref[c, :, :] = ad1[:, c * 4:(c + 1) * 4]


def _tc_a(x, W1, ms, md):
    blk = 1000
    return pl.pallas_call(
        _tc_a_body,
        grid=(NN // blk,),
        in_specs=[
            pl.BlockSpec((blk, 256), lambda i: (i, 0)),
            pl.BlockSpec((256, 64), lambda i: (0, 0)),
            pl.BlockSpec((64, 8), lambda i: (0, 0)),
            pl.BlockSpec((64, 8), lambda i: (0, 0)),
        ],
        out_specs=[
            pl.BlockSpec((blk, 4, T1W), lambda i: (i, 0, 0)),
            pl.BlockSpec((NC, blk, 4), lambda i: (0, i, 0)),
        ],
        out_shape=[
            jax.ShapeDtypeStruct((NN, 4, T1W), _f32),
            jax.ShapeDtypeStruct((NC, NN, 4), _f32),
        ],
    )(x, W1, ms, md)


def _tc_b_body(acc_ref, dn_ref, b1_ref, w2_ref, m2_ref, r_ref,
               h2_ref, aa_ref):
    msg = jnp.concatenate([acc_ref[k] for k in range(4)], axis=1)
    den = jnp.concatenate([dn_ref[0, :, 0:4], dn_ref[1, :, 0:4]], axis=1)
    denx = jnp.dot(den, r_ref[...], preferred_element_type=_f32)
    h = msg / denx + b1_ref[...]
    h = jnp.where(h > 0.0, h, jnp.exp(h) - 1.0)
    h2 = jnp.dot(h, w2_ref[...], preferred_element_type=_f32)
    aa = jnp.dot(h2, m2_ref[...], preferred_element_type=_f32)
    h2_ref[...] = h2
    aa_ref[...] = aa


def _tc_b(acc1, dn1, b1, W2, m2, rexp):
    blk = 1000
    return pl.pallas_call(
        _tc_b_body,
        grid=(NN // blk,),
        in_specs=[
            pl.BlockSpec((4, blk, C1W), lambda i: (0, i, 0)),
            pl.BlockSpec((NC, blk, 8), lambda i: (0, i, 0)),
            pl.BlockSpec((1, 64), lambda i: (0, 0)),
            pl.BlockSpec((64, 256), lambda i: (0, 0)),
            pl.BlockSpec((256, 2), lambda i: (0, 0)),
            pl.BlockSpec((8, 64), lambda i: (0, 0)),
        ],
        out_specs=[
            pl.BlockSpec((blk, 256), lambda i: (i, 0)),
            pl.BlockSpec((blk, 2), lambda i: (i, 0)),
        ],
        out_shape=[
            jax.ShapeDtypeStruct((NN, 256), _f32),
            jax.ShapeDtypeStruct((NN, 2), _f32),
        ],
    )(acc1, dn1, b1, W2, m2, rexp)


def _tc_c_body(acc_ref, da_ref, db_ref, b2_ref, o_ref):
    oa = jnp.concatenate([acc_ref[k] for k in range(4)], axis=1) / da_ref[...]
    ob = jnp.concatenate([acc_ref[k] for k in range(4, 8)],
                         axis=1) / db_ref[...]
    o = jnp.concatenate([oa, ob], axis=1) + b2_ref[...]
    m = jnp.max(o, axis=1, keepdims=True)
    lse = m + jnp.log(jnp.sum(jnp.exp(o - m), axis=1, keepdims=True))
    o_ref[...] = o - lse


def _tc_c(acc2, den_a, den_b, b2):
    blk = 1000
    return pl.pallas_call(
        _tc_c_body,
        grid=(NN // blk,),
        in_specs=[
            pl.BlockSpec((8, blk, C2W), lambda i: (0, i, 0)),
            pl.BlockSpec((blk, 1), lambda i: (i, 0)),
            pl.BlockSpec((blk, 1), lambda i: (i, 0)),
            pl.BlockSpec((1, 256), lambda i: (0, 0)),
        ],
        out_specs=pl.BlockSpec((blk, 256), lambda i: (i, 0)),
        out_shape=jax.ShapeDtypeStruct((NN, 256), _f32),
    )(acc2, den_a, den_b, b2)


def _zero_buf(msgbuf, width):
    offs = list(range(0, width - LN + 1, LN))
    if width % LN:
        offs.append(width - LN)

    def zrow(r, _):
        for o in offs:
            msgbuf[r, pl.ds(o, LN)] = jnp.zeros((LN,), _f32)
        return 0

    lax.fori_loop(0, NRC, zrow, 0)


def _zero_acc(msgbuf, acc_s, sub):
    for k in range(NROW // NRC):
        pltpu.sync_copy(msgbuf.at[pl.ds(0, NRC)],
                        acc_s.at[pl.ds(sub * NROW + k * NRC, NRC)])


def _copy_out(msgbuf, acc_s, out_hbm, core, sub):
    for k in range(NROW // NRC):
        rows = pl.ds(sub * NROW + k * NRC, NRC)
        pltpu.sync_copy(acc_s.at[rows], msgbuf.at[pl.ds(0, NRC)])
        pltpu.sync_copy(msgbuf.at[pl.ds(0, NRC)], out_hbm.at[core, rows])


def _mk_idx2(src_l, idx2_l, core):
    def mkidx(i, _):
        v = src_l[pl.ds(i * LN, LN)]
        idx2_l[pl.ds(i * LN, LN)] = v * 2 + core
        return 0

    lax.fori_loop(0, CHUNK // LN, mkidx, 0)


def _sc1_body(t1_hbm, ad_hbm, src_hbm, dst_hbm, out_hbm, dn_hbm,
              adt_l, src_l, idx2_l, dst_l, rb0, rb1, rb2, mb0, mb1, mb2,
              pbuf, denbuf, acc_s, den_s,
              gs0, gs1, gs2, ss0, ss1, ss2, dsem):
    rbufs = [rb0, rb1, rb2]
    mbufs = [mb0, mb1, mb2]
    gsems = [gs0, gs1, gs2]
    ssems = [ss0, ss1, ss2]
    msgbuf = mb0
    core = lax.axis_index("c")
    sub = lax.axis_index("s")
    iota = lax.iota(_i32, LN)

    def zdb(i, _):
        w = i * LN + iota
        plsc.store_scatter(denbuf, [w // 8, w % 8], jnp.zeros((LN,), _f32))
        return 0

    lax.fori_loop(0, BB * 8 // LN, zdb, 0)
    for k in range(NROW // NRC):
        pltpu.sync_copy(denbuf.at[pl.ds(0, NRC)],
                        den_s.at[pl.ds(sub * NROW + k * NRC, NRC)])
    pltpu.sync_copy(ad_hbm.at[core], adt_l)
    ov1 = jnp.where(iota >= 8, 1, 0)
    om2 = iota % 2
    hsel = iota // 2

    def half(hh, _):
        _zero_buf(msgbuf, C1W)
        _zero_acc(msgbuf, acc_s, sub)
        plsc.subcore_barrier()

        def chunk(ci, _):
            g = sub * 2 + ci
            pltpu.sync_copy(src_hbm.at[pl.ds(g * CHUNK, CHUNK)], src_l)
            pltpu.sync_copy(dst_hbm.at[pl.ds(g * CHUNK, CHUNK)], dst_l)
            base = g * CHUNK

            def mkidx(i, _):
                v = src_l[pl.ds(i * LN, LN)]
                idx2_l[pl.ds(i * LN, LN)] = v * 4 + core * 2 + hh
                return 0

            lax.fori_loop(0, CHUNK // LN, mkidx, 0)

            for k in range(2):
                pltpu.async_copy(
                    t1_hbm.at[idx2_l.at[pl.ds(k * BB, BB)]],
                    rbufs[k], gsems[k])

            def run_batch(b, i, k):
                rb = rbufs[k]
                mb = mbufs[k]
                db = denbuf
                dsl = dst_l.at[pl.ds(b * BB, BB)]

                @pl.when(i > 0)
                def _():
                    pltpu.make_async_copy(mb, acc_s.at[dsl],
                                          ssems[k]).wait()

                @pl.when(jnp.logical_or(i > 0, k > 0))
                def _():
                    pltpu.make_async_copy(db, den_s.at[dsl],
                                          dsem).wait()

                pltpu.make_async_copy(
                    t1_hbm.at[idx2_l.at[pl.ds(b * BB, BB)]], rb,
                    gsems[k]).wait()

                @pl.when(b + 2 < NBATCH)
                def _():
                    pltpu.async_copy(
                        t1_hbm.at[idx2_l.at[pl.ds((b + 2) * BB, BB)]],
                        rbufs[(k + 2) % 3], gsems[(k + 2) % 3])

                def group(g2, _):
                    dstv = dst_l[pl.ds(b * BB + g2 * LN, LN)]
                    rowi = g2 * LN + iota
                    valid = (base + b * BB + rowi) < ET
                    for h in range(2):
                        hf = jnp.full((LN,), h, _i32)
                        a_s = plsc.load_gather(rb, [rowi, hf + 16])
                        a_d = plsc.load_gather(adt_l,
                                               [dstv, hf + hh * 2])
                        al = a_s + a_d
                        al = jnp.where(al > 0.0, al, 0.2 * al)
                        p = jnp.where(valid, jnp.exp(al), 0.0)
                        plsc.store_scatter(pbuf, [iota, hf], p)
                    for e in range(16):
                        er = g2 * LN + e
                        ef = jnp.full((LN,), e, _i32)
                        pe = plsc.load_gather(pbuf, [ef, ov1])
                        mb[er, pl.ds(0, LN)] = (
                            rb[er, pl.ds(0, LN)] * pe)
                        pt = plsc.load_gather(pbuf, [ef, om2])
                        pt = jnp.where(hsel == hh, pt, 0.0)
                        plsc.store_scatter(
                            db, [jnp.full((LN,), er, _i32), iota],
                            pt, mask=iota < 8)
                    return 0

                lax.fori_loop(0, BB // LN, group, 0)
                pltpu.async_copy(mb, acc_s.at[dsl], ssems[k], add=True)
                pltpu.async_copy(db, den_s.at[dsl], dsem, add=True)

            def batch3(i, _):
                for k in range(3):
                    run_batch(i * 3 + k, i, k)
                return 0

            lax.fori_loop(0, NBATCH // 3, batch3, 0)
            for k in range(3):
                dsl = dst_l.at[pl.ds((NBATCH - 3 + k) * BB, BB)]
                pltpu.make_async_copy(mbufs[k], acc_s.at[dsl],
                                      ssems[k]).wait()
            dsl = dst_l.at[pl.ds((NBATCH - 1) * BB, BB)]
            pltpu.make_async_copy(denbuf, den_s.at[dsl], dsem).wait()
            return 0

        lax.fori_loop(0, 2, chunk, 0)
        plsc.subcore_barrier()
        _copy_out(msgbuf, acc_s, out_hbm, core * 2 + hh, sub)
        plsc.subcore_barrier()
        return 0

    lax.fori_loop(0, 2, half, 0)
    for k in range(NROW // NRC):
        rows = pl.ds(sub * NROW + k * NRC, NRC)
        pltpu.sync_copy(den_s.at[rows], denbuf.at[pl.ds(0, NRC)])
        pltpu.sync_copy(denbuf.at[pl.ds(0, NRC)], dn_hbm.at[core, rows])


def _sc_phase1(t1pk, adt, srcg, dstg):
    mesh = plsc.VectorSubcoreMesh(
        core_axis_name="c", subcore_axis_name="s",
        num_cores=NC, num_subcores=NS)
    f = functools.partial(
        pl.kernel,
        out_type=[
            jax.ShapeDtypeStruct((4, NN, C1W), _f32),
            jax.ShapeDtypeStruct((NC, NN, 8), _f32),
        ],
        mesh=mesh,
        compiler_params=pltpu.CompilerParams(
            needs_layout_passes=False, use_tc_tiling_on_sc=False),
        scratch_types=[
            pltpu.VMEM((NN, 4), _f32),
            pltpu.VMEM((CHUNK,), _i32),
            pltpu.VMEM((CHUNK,), _i32),
            pltpu.VMEM((CHUNK,), _i32),
            pltpu.VMEM((BB, T1W), _f32),
            pltpu.VMEM((BB, T1W), _f32),
            pltpu.VMEM((BB, T1W), _f32),
            pltpu.VMEM((BB, C1W), _f32),
            pltpu.VMEM((BB, C1W), _f32),
            pltpu.VMEM((BB, C1W), _f32),
            pltpu.VMEM((LN, 2), _f32),
            pltpu.VMEM((BB, 8), _f32),
            pltpu.VMEM_SHARED((NN, C1W), _f32),
            pltpu.VMEM_SHARED((NN, 8), _f32),
        ] + [pltpu.SemaphoreType.DMA] * 7,
    )(_sc1_body)
    return f(t1pk, adt, srcg, dstg)


def _sc2_body(h2_hbm, as2_hbm, ad2_hbm, src_hbm, dst_hbm, out_hbm, dn_hbm,
              as2_l, ad2_l, src_l, idx2_l, dst_l, rb0, rb1, rb2,
              mb0, mb1, mb2, qbuf, den_l, idr_l, acc_s, den_s,
              gs0, gs1, gs2, ss0, ss1, ss2):
    rbufs = [rb0, rb1, rb2]
    mbufs = [mb0, mb1, mb2]
    gsems = [gs0, gs1, gs2]
    ssems = [ss0, ss1, ss2]
    msgbuf = mb0
    core = lax.axis_index("c")
    sub = lax.axis_index("s")
    iota = lax.iota(_i32, LN)

    def zden(i, _):
        den_l[pl.ds(i * LN, LN)] = jnp.zeros((LN,), _f32)
        return 0

    lax.fori_loop(0, NN // LN, zden, 0)

    @pl.when(sub == 0)
    def _():
        pltpu.sync_copy(den_l, den_s)

    pltpu.sync_copy(as2_hbm, as2_l)
    pltpu.sync_copy(ad2_hbm, ad2_l)

    def half(hh, _):
        _zero_buf(msgbuf, C2W)
        _zero_acc(msgbuf, acc_s, sub)
        plsc.subcore_barrier()

        def chunk(ci, _):
            g = sub * 2 + ci
            pltpu.sync_copy(src_hbm.at[pl.ds(g * CHUNK, CHUNK)], src_l)
            pltpu.sync_copy(dst_hbm.at[pl.ds(g * CHUNK, CHUNK)], dst_l)
            base = g * CHUNK

            def mkidx(i, _):
                v = src_l[pl.ds(i * LN, LN)]
                idx2_l[pl.ds(i * LN, LN)] = v * 8 + core * 4 + hh
                return 0

            lax.fori_loop(0, CHUNK // LN, mkidx, 0)

            for k in range(2):
                pltpu.async_copy(
                    h2_hbm.at[idx2_l.at[pl.ds(k * BB, BB)]],
                    rbufs[k], gsems[k])

            def run_batch(b, i, k):
                rb = rbufs[k]
                mb = mbufs[k]
                dsl = dst_l.at[pl.ds(b * BB, BB)]

                @pl.when(i > 0)
                def _():
                    pltpu.make_async_copy(mb, acc_s.at[dsl],
                                          ssems[k]).wait()

                pltpu.make_async_copy(
                    h2_hbm.at[idx2_l.at[pl.ds(b * BB, BB)]], rb,
                    gsems[k]).wait()

                @pl.when(b + 2 < NBATCH)
                def _():
                    pltpu.async_copy(
                        h2_hbm.at[idx2_l.at[pl.ds((b + 2) * BB, BB)]],
                        rbufs[(k + 2) % 3], gsems[(k + 2) % 3])

                def group(g2, _):
                    srcv = src_l[pl.ds(b * BB + g2 * LN, LN)]
                    dstv = dst_l[pl.ds(b * BB + g2 * LN, LN)]
                    rowi = g2 * LN + iota
                    valid = (base + b * BB + rowi) < ET
                    a_s = plsc.load_gather(as2_l, [srcv])
                    a_d = plsc.load_gather(ad2_l, [dstv])
                    al = a_s + a_d
                    al = jnp.where(al > 0.0, al, 0.2 * al)
                    q = jnp.where(valid, jnp.exp(al), 0.0)
                    qbuf[...] = q
                    for e in range(16):
                        er = g2 * LN + e
                        qe = plsc.load_gather(
                            qbuf, [jnp.full((LN,), e, _i32)])
                        for j in range(C2W // LN):
                            mb[er, pl.ds(j * LN, LN)] = (
                                rb[er, pl.ds(j * LN, LN)] * qe)

                    @pl.when(hh == 0)
                    def _():
                        for e in range(16):
                            plsc.addupdate_scatter(den_l, [dstv], q,
                                                   mask=iota == e)

                    return 0

                lax.fori_loop(0, BB // LN, group, 0)
                pltpu.async_copy(mb, acc_s.at[dsl], ssems[k], add=True)

            def batch3(i, _):
                for k in range(3):
                    run_batch(i * 3 + k, i, k)
                return 0

            lax.fori_loop(0, NBATCH // 3, batch3, 0)
            for k in range(3):
                dsl = dst_l.at[pl.ds((NBATCH - 3 + k) * BB, BB)]
                pltpu.make_async_copy(mbufs[k], acc_s.at[dsl],
                                      ssems[k]).wait()
            return 0

        lax.fori_loop(0, 2, chunk, 0)
        plsc.subcore_barrier()
        _copy_out(msgbuf, acc_s, out_hbm, core * 4 + hh, sub)
        plsc.subcore_barrier()
        return 0

    lax.fori_loop(0, 4, half, 0)

    def dred(k, _):
        rbase = k * BB
        for t in range(BB // LN):
            idr_l[pl.ds(t * LN, LN)] = rbase + t * LN + iota
        pltpu.sync_copy(den_l.at[pl.ds(rbase, BB)], den_s.at[idr_l],
                        add=True)
        return 0

    lax.fori_loop(0, NN // BB, dred, 0)
    plsc.subcore_barrier()

    @pl.when(sub == 0)
    def _():
        pltpu.sync_copy(den_s, den_l)
        pltpu.sync_copy(den_l, dn_hbm.at[core])


def _sc_phase2(h2pk, as2, ad2, srcg, dstg):
    mesh = plsc.VectorSubcoreMesh(
        core_axis_name="c", subcore_axis_name="s",
        num_cores=NC, num_subcores=NS)
    f = functools.partial(
        pl.kernel,
        out_type=[
            jax.ShapeDtypeStruct((8, NN, C2W), _f32),
            jax.ShapeDtypeStruct((NC, NN), _f32),
        ],
        mesh=mesh,
        compiler_params=pltpu.CompilerParams(
            needs_layout_passes=False, use_tc_tiling_on_sc=False),
        scratch_types=[
            pltpu.VMEM((NN,), _f32),
            pltpu.VMEM((NN,), _f32),
            pltpu.VMEM((CHUNK,), _i32),
            pltpu.VMEM((CHUNK,), _i32),
            pltpu.VMEM((CHUNK,), _i32),
            pltpu.VMEM((BB, C2W), _f32),
            pltpu.VMEM((BB, C2W), _f32),
            pltpu.VMEM((BB, C2W), _f32),
            pltpu.VMEM((BB, C2W), _f32),
            pltpu.VMEM((BB, C2W), _f32),
            pltpu.VMEM((BB, C2W), _f32),
            pltpu.VMEM((LN,), _f32),
            pltpu.VMEM((NN,), _f32),
            pltpu.VMEM((BB,), _i32),
            pltpu.VMEM_SHARED((NN, C2W), _f32),
            pltpu.VMEM_SHARED((NN,), _f32),
        ] + [pltpu.SemaphoreType.DMA] * 6,
    )(_sc2_body)
    return f(h2pk, as2, ad2, srcg, dstg)


@jax.jit
def kernel(x, edge_index, W1, a_src1, a_dst1, b1, W2, a_src2, a_dst2, b2):
    loop = jnp.arange(NN, dtype=_i32)
    src = jnp.concatenate([edge_index[0].astype(_i32), loop])
    dst = jnp.concatenate([edge_index[1].astype(_i32), loop])
    pad = jnp.zeros((EP - ET,), _i32)
    srcg = jnp.concatenate([src, pad])
    dstg = jnp.concatenate([dst, pad])

    diag = jnp.repeat(jnp.eye(8, dtype=_f32), 8, axis=0)
    ms = diag * a_src1.reshape(64)[:, None]
    md = diag * a_dst1.reshape(64)[:, None]
    rexp = jnp.repeat(jnp.eye(8, dtype=_f32), 8, axis=1)
    m2 = jnp.stack([a_src2.reshape(256), a_dst2.reshape(256)], axis=1)

    t1, adt = _tc_a(x, W1, ms, md)
    t1pk = t1.reshape(4 * NN, T1W)
    acc1, dn1 = _sc_phase1(t1pk, adt, srcg, dstg)
    h2, aa = _tc_b(acc1, dn1, b1.reshape(1, 64), W2, m2, rexp)
    h2pk = h2.reshape(8 * NN, C2W)
    acc2, den2 = _sc_phase2(h2pk, aa[:, 0], aa[:, 1], srcg, dstg)
    return _tc_c(acc2, den2[0].reshape(NN, 1),
                 den2[1].reshape(NN, 1), b2.reshape(1, 256))
